# baseline (speedup 1.0000x reference)
import functools
import math

import jax
import jax.numpy as jnp
from jax import lax
from jax.experimental import pallas as pl
from jax.experimental.pallas import tpu as pltpu

_RATIO = 0.9
_W = 12


def _stats_kernel(x_hbm, maxstd_ref, means_ref, xbuf, stdacc, sem_in,
                  *, B, T, D, rem, W, nw_t, tiles, rows_a):
    rows_t = nw_t * W
    nsteps = tiles * B

    def tile_start(i):
        return min(((rem + i * rows_t) // 8) * 8, T - rows_a)

    def in_copy(s, slot):
        i, b = divmod(s, B)
        return pltpu.make_async_copy(
            x_hbm.at[b, pl.ds(tile_start(i), rows_a), :],
            xbuf.at[slot],
            sem_in.at[slot],
        )

    in_copy(0, 0).start()
    maxvals = []
    for s in range(nsteps):
        slot = s % 2
        i, b = divmod(s, B)
        if s + 1 < nsteps:
            in_copy(s + 1, (s + 1) % 2).start()
        in_copy(s, slot).wait()
        xb = xbuf[slot]
        mu = jnp.mean(xb, axis=1, keepdims=True)
        dd = xb - mu
        var = jnp.sum(dd * dd, axis=1, keepdims=True) * (1.0 / (D - 1))
        std = jnp.sqrt(var)
        if b == 0:
            stdacc[...] = std
        else:
            stdacc[...] = stdacc[...] + std

        a0 = tile_start(i)
        tok_r = lax.broadcasted_iota(jnp.int32, (nw_t, rows_a), 1) + a0
        w_id = lax.broadcasted_iota(jnp.int32, (nw_t, rows_a), 0) + i * nw_t
        valid = (tok_r >= rem) & ((tok_r - rem) // W == w_id)
        onehot = valid.astype(jnp.float32)
        wm = jnp.dot(onehot, xb, preferred_element_type=jnp.float32) * (1.0 / W)
        means_ref[b, i] = wm

        if b == B - 1:
            acc = stdacc[...] * (1.0 / B)
            tok_c = lax.broadcasted_iota(jnp.int32, (rows_a, nw_t), 0) + a0
            w_id2 = lax.broadcasted_iota(jnp.int32, (rows_a, nw_t), 1) + i * nw_t
            mask = (tok_c >= rem) & ((tok_c - rem) // W == w_id2)
            big = jnp.where(mask, jnp.broadcast_to(acc, (rows_a, nw_t)),
                            -jnp.inf)
            maxvals.append(jnp.max(big, axis=0))

    maxstd_ref[...] = jnp.stack(maxvals, axis=0)


def _select_kernel(mcol_ref, mrow_ref, sel_ref, off_ref, *, NW, k, W):
    vc = mcol_ref[...]
    vr = mrow_ref[...]
    ii = lax.broadcasted_iota(jnp.int32, (NW, NW), 0)
    jj = lax.broadcasted_iota(jnp.int32, (NW, NW), 1)
    beats = ((vr < vc) | ((vr == vc) & (jj < ii))).astype(jnp.float32)
    rank_c = jnp.sum(beats, axis=1, keepdims=True)
    sel_c = rank_c < k
    rank_r = (NW - 1) - jnp.sum(beats, axis=0, keepdims=True)
    size_r = jnp.where(rank_r < k, 1.0, float(W))
    jlt = jj < ii
    off_c = jnp.sum(jnp.where(jlt, jnp.broadcast_to(size_r, (NW, NW)), 0.0),
                    axis=1, keepdims=True)
    sel_ref[...] = sel_c.astype(jnp.int32)
    off_ref[...] = off_c.astype(jnp.int32)


def _compact_kernel(x_hbm, means_hbm, sel_ref, off_ref, out_hbm, sems, sem0,
                    *, B, T, D, rem, W, NW, L, NRING):
    OL = rem + L

    for b in range(B):
        pltpu.make_async_copy(x_hbm.at[pl.ds(b * T, rem)],
                              out_hbm.at[pl.ds(b * OL, rem)], sem0).start()

    def win_copy(b, w):
        ow = off_ref[w]
        return pltpu.make_async_copy(
            x_hbm.at[pl.ds(b * T + rem + w * W, W)],
            out_hbm.at[pl.ds(b * OL + rem + ow, W)],
            sems.at[lax.rem(b * NW + w, NRING)])

    def mean_copy(b, w):
        ow = off_ref[w]
        return pltpu.make_async_copy(
            means_hbm.at[pl.ds(b * NW + w, 1)],
            out_hbm.at[pl.ds(b * OL + rem + ow, 1)],
            sems.at[lax.rem(b * NW + w, NRING)])

    def issue(u):
        b = u // NW
        w = u - b * NW
        sw = sel_ref[w]

        @pl.when(sw == 0)
        def _():
            win_copy(b, w).start()

        @pl.when(sw != 0)
        def _():
            mean_copy(b, w).start()

    def wait_u(u):
        b = u // NW
        w = u - b * NW
        sw = sel_ref[w]

        @pl.when(sw == 0)
        def _():
            win_copy(b, w).wait()

        @pl.when(sw != 0)
        def _():
            mean_copy(b, w).wait()

    def body(u, carry):
        @pl.when(u >= NRING)
        def _():
            wait_u(u - NRING)

        issue(u)
        return carry

    lax.fori_loop(0, B * NW, body, 0)
    for t in range(NRING):
        wait_u(B * NW - NRING + t)
    for b in range(B):
        pltpu.make_async_copy(x_hbm.at[pl.ds(b * T, rem)],
                              out_hbm.at[pl.ds(b * OL, rem)], sem0).wait()


def kernel(x):
    B, T, D = x.shape
    W = _W
    k = math.floor((T - T * _RATIO) / W)
    rem = T % W
    NW = (T - rem) // W
    L = NW * W - k * (W - 1)
    nw_t = max(d for d in range(1, min(NW, 32) + 1) if NW % d == 0)
    tiles = NW // nw_t
    rows_a = ((nw_t * W + rem) + 7) // 8 * 8

    maxstd, means = pl.pallas_call(
        functools.partial(_stats_kernel, B=B, T=T, D=D, rem=rem, W=W,
                          nw_t=nw_t, tiles=tiles, rows_a=rows_a),
        in_specs=[pl.BlockSpec(memory_space=pl.ANY)],
        out_specs=[pl.BlockSpec(memory_space=pltpu.VMEM),
                   pl.BlockSpec(memory_space=pltpu.VMEM)],
        out_shape=[jax.ShapeDtypeStruct((tiles, nw_t), jnp.float32),
                   jax.ShapeDtypeStruct((B, tiles, nw_t, D), jnp.float32)],
        scratch_shapes=[
            pltpu.VMEM((2, rows_a, D), jnp.float32),
            pltpu.VMEM((rows_a, 1), jnp.float32),
            pltpu.SemaphoreType.DMA((2,)),
        ],
    )(x)

    m = maxstd.reshape(NW)
    sel2, off2 = pl.pallas_call(
        functools.partial(_select_kernel, NW=NW, k=k, W=W),
        out_shape=[jax.ShapeDtypeStruct((NW, 1), jnp.int32),
                   jax.ShapeDtypeStruct((NW, 1), jnp.int32)],
    )(m[:, None], m[None, :])

    sel = sel2.reshape(NW)
    off = off2.reshape(NW)

    NRING = 8
    out = pl.pallas_call(
        functools.partial(_compact_kernel, B=B, T=T, D=D, rem=rem, W=W,
                          NW=NW, L=L, NRING=NRING),
        in_specs=[pl.BlockSpec(memory_space=pl.ANY),
                  pl.BlockSpec(memory_space=pl.ANY),
                  pl.BlockSpec(memory_space=pltpu.SMEM),
                  pl.BlockSpec(memory_space=pltpu.SMEM)],
        out_specs=pl.BlockSpec(memory_space=pl.ANY),
        out_shape=jax.ShapeDtypeStruct((B * (rem + L), 1, D), x.dtype),
        scratch_shapes=[
            pltpu.SemaphoreType.DMA((NRING,)),
            pltpu.SemaphoreType.DMA,
        ],
    )(x.reshape(B * T, 1, D), means.reshape(B * NW, 1, D), sel, off)
    return out.reshape(B, rem + L, D)


# speedup vs baseline: 11.6935x; 11.6935x over previous
import functools
import math

import jax
import jax.numpy as jnp
from jax import lax
from jax.experimental import pallas as pl
from jax.experimental.pallas import tpu as pltpu
from jax.experimental.pallas import tpu_sc as plsc

_RATIO = 0.9
_W = 12

_NC = 2
_NS = 16


def _stats_kernel(x_hbm, maxstd_ref, means_ref, xbuf, stdacc, sem_in,
                  *, B, T, D, rem, W, nw_t, tiles, rows_a):
    rows_t = nw_t * W
    nsteps = tiles * B

    def tile_start(i):
        return min(((rem + i * rows_t) // 8) * 8, T - rows_a)

    def in_copy(s, slot):
        i, b = divmod(s, B)
        return pltpu.make_async_copy(
            x_hbm.at[b, pl.ds(tile_start(i), rows_a), :],
            xbuf.at[slot],
            sem_in.at[slot],
        )

    in_copy(0, 0).start()
    maxvals = []
    for s in range(nsteps):
        slot = s % 2
        i, b = divmod(s, B)
        if s + 1 < nsteps:
            in_copy(s + 1, (s + 1) % 2).start()
        in_copy(s, slot).wait()
        xb = xbuf[slot]
        mu = jnp.mean(xb, axis=1, keepdims=True)
        dd = xb - mu
        var = jnp.sum(dd * dd, axis=1, keepdims=True) * (1.0 / (D - 1))
        std = jnp.sqrt(var)
        if b == 0:
            stdacc[...] = std
        else:
            stdacc[...] = stdacc[...] + std

        a0 = tile_start(i)
        tok_r = lax.broadcasted_iota(jnp.int32, (nw_t, rows_a), 1) + a0
        w_id = lax.broadcasted_iota(jnp.int32, (nw_t, rows_a), 0) + i * nw_t
        valid = (tok_r >= rem) & ((tok_r - rem) // W == w_id)
        onehot = valid.astype(jnp.float32)
        wm = jnp.dot(onehot, xb, preferred_element_type=jnp.float32) * (1.0 / W)
        means_ref[b, i] = wm

        if b == B - 1:
            acc = stdacc[...] * (1.0 / B)
            tok_c = lax.broadcasted_iota(jnp.int32, (rows_a, nw_t), 0) + a0
            w_id2 = lax.broadcasted_iota(jnp.int32, (rows_a, nw_t), 1) + i * nw_t
            mask = (tok_c >= rem) & ((tok_c - rem) // W == w_id2)
            big = jnp.where(mask, jnp.broadcast_to(acc, (rows_a, nw_t)),
                            -jnp.inf)
            maxvals.append(jnp.max(big, axis=0))

    maxstd_ref[...] = jnp.stack(maxvals, axis=0)


def _select_kernel(mcol_ref, mrow_ref, src_ref, mdst_ref, msrc_ref,
                   *, B, T, NW, k, W, rem, L, NPAD, HALF0):
    vc = mcol_ref[...]
    vr = mrow_ref[...]
    ii = lax.broadcasted_iota(jnp.int32, (NW, NW), 0)
    jj = lax.broadcasted_iota(jnp.int32, (NW, NW), 1)
    beats = ((vr < vc) | ((vr == vc) & (jj < ii))).astype(jnp.float32)
    rank_c = jnp.sum(beats, axis=1, keepdims=True)
    sel_c = (rank_c < k).astype(jnp.float32)
    size_c = jnp.where(sel_c > 0.5, 1.0, float(W))
    rank_r = (NW - 1) - jnp.sum(beats, axis=0, keepdims=True)
    sel_r = (rank_r < k).astype(jnp.float32)
    size_r = jnp.where(sel_r > 0.5, 1.0, float(W))
    size_rb = jnp.broadcast_to(size_r, (NW, NW))
    size_cb0 = jnp.broadcast_to(size_c, (NW, NW))
    sel_cb0 = jnp.broadcast_to(sel_c, (NW, NW))
    off_c = jnp.sum(jnp.where(jj < ii, size_rb, 0.0), axis=1, keepdims=True)
    off_row = jnp.sum(jnp.where(ii < jj, size_cb0, 0.0), axis=0, keepdims=True)
    c_row = jnp.sum(jnp.where(ii < jj, sel_cb0, 0.0), axis=0, keepdims=True)

    OL = rem + L
    pp = lax.broadcasted_iota(jnp.int32, (NW, OL), 1).astype(jnp.float32)
    ww = lax.broadcasted_iota(jnp.int32, (NW, OL), 0).astype(jnp.float32)
    ll = pp - float(rem)
    off_cb = jnp.broadcast_to(off_c, (NW, OL))
    size_cb = jnp.broadcast_to(size_c, (NW, OL))
    inwin = ((ll >= off_cb) & (ll < off_cb + size_cb)).astype(jnp.float32)
    contrib = inwin * (float(rem) + float(W) * ww + (ll - off_cb))
    f_row = jnp.sum(contrib, axis=0, keepdims=True)
    p1 = lax.broadcasted_iota(jnp.int32, (1, OL), 1).astype(jnp.float32)
    f_row = jnp.where(p1 < rem, p1, f_row)
    bb = lax.broadcasted_iota(jnp.int32, (B, OL), 0).astype(jnp.float32)
    src = f_row + bb * float(T)
    src_ref[...] = src.astype(jnp.int32)

    jjf = lax.broadcasted_iota(jnp.int32, (NPAD, 1), 0).astype(jnp.float32)
    b_j = jnp.floor((jjf + 0.5) * (1.0 / k))
    t_j = jjf - b_j * float(k)
    selr_b = jnp.broadcast_to(sel_r, (NPAD, NW))
    cr_b = jnp.broadcast_to(c_row, (NPAD, NW))
    offr_b = jnp.broadcast_to(off_row, (NPAD, NW))
    wr = lax.broadcasted_iota(jnp.int32, (NPAD, NW), 1).astype(jnp.float32)
    hit = selr_b * (cr_b == jnp.broadcast_to(t_j, (NPAD, NW))).astype(jnp.float32)
    dst_col = jnp.sum(hit * (float(rem) + offr_b), axis=1, keepdims=True) \
        + b_j * float(OL)
    src_col = jnp.sum(hit * wr, axis=1, keepdims=True) + b_j * float(NW)
    valid_col = (jjf < float(B * k)).astype(jnp.float32)
    dst_col = jnp.where(valid_col > 0.5, dst_col, -1.0)
    src_col = jnp.where(valid_col > 0.5, src_col, 0.0)

    c_col = jnp.sum(jnp.where(jj < ii, jnp.broadcast_to(sel_r, (NW, NW)), 0.0),
                    axis=1, keepdims=True)
    qq = lax.broadcasted_iota(jnp.int32, (1, NPAD), 1).astype(jnp.float32)
    b_q = jnp.floor((qq + 0.5) * (1.0 / k))
    t_q = qq - b_q * float(k)
    selc_b = jnp.broadcast_to(sel_c, (NW, NPAD))
    cc_b = jnp.broadcast_to(c_col, (NW, NPAD))
    offc_b = jnp.broadcast_to(off_c, (NW, NPAD))
    wc = lax.broadcasted_iota(jnp.int32, (NW, NPAD), 0).astype(jnp.float32)
    hitT = selc_b * (cc_b == jnp.broadcast_to(t_q, (NW, NPAD))).astype(jnp.float32)
    dst_row = jnp.sum(hitT * (float(rem) + offc_b), axis=0, keepdims=True) \
        + b_q * float(OL)
    src_row = jnp.sum(hitT * wc, axis=0, keepdims=True) + b_q * float(NW)
    valid_row = (qq < float(B * k)).astype(jnp.float32)
    dst_row = jnp.where(valid_row > 0.5, dst_row, -1.0)
    src_row = jnp.where(valid_row > 0.5, src_row, 0.0)

    ii2 = lax.broadcasted_iota(jnp.int32, (NPAD, NPAD), 0)
    jj2 = lax.broadcasted_iota(jnp.int32, (NPAD, NPAD), 1)
    rows_dst, rows_src = [], []
    for h in range(2):
        if h == 0:
            inh_col = ((dst_col >= 0.0) & (dst_col < float(HALF0))
                       ).astype(jnp.float32)
            inh_row = ((dst_row >= 0.0) & (dst_row < float(HALF0))
                       ).astype(jnp.float32)
        else:
            inh_col = (dst_col >= float(HALF0)).astype(jnp.float32)
            inh_row = (dst_row >= float(HALF0)).astype(jnp.float32)
        r_col = jnp.sum(jnp.where(jj2 < ii2,
                                  jnp.broadcast_to(inh_row, (NPAD, NPAD)),
                                  0.0), axis=1, keepdims=True)
        m_jq = (jnp.broadcast_to(inh_col, (NPAD, NPAD)) *
                (jnp.broadcast_to(r_col, (NPAD, NPAD)) ==
                 jnp.broadcast_to(qq.astype(jnp.float32), (NPAD, NPAD))
                 ).astype(jnp.float32))
        dstl = jnp.sum(m_jq * jnp.broadcast_to(dst_col, (NPAD, NPAD)),
                       axis=0, keepdims=True)
        srcl = jnp.sum(m_jq * jnp.broadcast_to(src_col, (NPAD, NPAD)),
                       axis=0, keepdims=True)
        anyq = jnp.sum(m_jq, axis=0, keepdims=True)
        first_mask = inh_col * (r_col == 0.0).astype(jnp.float32)
        fall_dst = jnp.sum(first_mask * dst_col, axis=0, keepdims=True)
        fall_src = jnp.sum(first_mask * src_col, axis=0, keepdims=True)
        rows_dst.append(jnp.where(anyq > 0.5, dstl,
                                  jnp.broadcast_to(fall_dst, (1, NPAD))))
        rows_src.append(jnp.where(anyq > 0.5, srcl,
                                  jnp.broadcast_to(fall_src, (1, NPAD))))
    mdst_ref[...] = jnp.concatenate(rows_dst, axis=0).astype(jnp.int32)
    msrc_ref[...] = jnp.concatenate(rows_src, axis=0).astype(jnp.int32)


def _sc_compact(x_hbm, src_hbm, means_hbm, mdst_hbm, msrc_hbm, out_hbm,
                idxv, gbuf, mdstv, msrcv, mbuf, gsem, wsem, msem,
                *, TOTAL, D, CHUNK, STRIDE, HALF0, NSUB, NPAD):
    c = lax.axis_index("c")
    s = lax.axis_index("s")
    base = c * HALF0
    cap = jnp.where(c == 0, HALF0 - CHUNK, TOTAL - HALF0 - CHUNK)
    start = base + jnp.minimum(STRIDE * s, cap)
    start = pl.multiple_of(start, 8)

    pltpu.sync_copy(src_hbm.at[pl.ds(start, CHUNK)], idxv)

    SUB = CHUNK // NSUB

    def gather(it, slot):
        return pltpu.make_async_copy(
            x_hbm.at[idxv.at[pl.ds(it * SUB, SUB)]],
            gbuf.at[slot], gsem.at[slot])

    def write(it, slot):
        return pltpu.make_async_copy(
            gbuf.at[slot],
            out_hbm.at[pl.ds(start + it * SUB, SUB), :], wsem.at[slot])

    gather(0, 0).start()
    for it in range(NSUB):
        slot = it % 2
        if it >= 1:
            write(it - 1, (it - 1) % 2).wait()
        if it + 1 < NSUB:
            gather(it + 1, (it + 1) % 2).start()
        gather(it, slot).wait()
        write(it, slot).start()
    write(NSUB - 1, (NSUB - 1) % 2).wait()

    plsc.subcore_barrier()

    off_e = pl.multiple_of(c * NPAD + s * 16, 8)
    pltpu.sync_copy(mdst_hbm.at[pl.ds(off_e, 16)], mdstv)
    pltpu.sync_copy(msrc_hbm.at[pl.ds(off_e, 16)], msrcv)
    gm = pltpu.make_async_copy(means_hbm.at[msrcv], mbuf, msem)
    gm.start()
    gm.wait()
    sm = pltpu.make_async_copy(mbuf, out_hbm.at[mdstv], msem)
    sm.start()
    sm.wait()


def kernel(x):
    B, T, D = x.shape
    W = _W
    k = math.floor((T - T * _RATIO) / W)
    rem = T % W
    NW = (T - rem) // W
    L = NW * W - k * (W - 1)
    OL = rem + L
    nw_t = max(d for d in range(1, min(NW, 32) + 1) if NW % d == 0)
    tiles = NW // nw_t
    rows_a = ((nw_t * W + rem) + 7) // 8 * 8

    maxstd, means = pl.pallas_call(
        functools.partial(_stats_kernel, B=B, T=T, D=D, rem=rem, W=W,
                          nw_t=nw_t, tiles=tiles, rows_a=rows_a),
        in_specs=[pl.BlockSpec(memory_space=pl.ANY)],
        out_specs=[pl.BlockSpec(memory_space=pltpu.VMEM),
                   pl.BlockSpec(memory_space=pltpu.VMEM)],
        out_shape=[jax.ShapeDtypeStruct((tiles, nw_t), jnp.float32),
                   jax.ShapeDtypeStruct((B, tiles, nw_t, D), jnp.float32)],
        scratch_shapes=[
            pltpu.VMEM((2, rows_a, D), jnp.float32),
            pltpu.VMEM((rows_a, 1), jnp.float32),
            pltpu.SemaphoreType.DMA((2,)),
        ],
    )(x)

    NPAD = _NS * 16
    TOTAL = B * OL
    CHUNK = 512
    HALF0 = (TOTAL // 2 + 7) // 8 * 8
    m = maxstd.reshape(NW)
    src_all, mdst, msrc = pl.pallas_call(
        functools.partial(_select_kernel, B=B, T=T, NW=NW, k=k, W=W,
                          rem=rem, L=L, NPAD=NPAD, HALF0=HALF0),
        out_shape=[jax.ShapeDtypeStruct((B, OL), jnp.int32),
                   jax.ShapeDtypeStruct((2, NPAD), jnp.int32),
                   jax.ShapeDtypeStruct((2, NPAD), jnp.int32)],
    )(m[:, None], m[None, :])
    span = max(HALF0, TOTAL - HALF0)
    STRIDE = max(8, (span - CHUNK + (_NS - 1) * 8 - 1) // ((_NS - 1) * 8) * 8)
    NSUB = CHUNK // 32

    mesh = plsc.VectorSubcoreMesh(core_axis_name="c", subcore_axis_name="s")
    sck = functools.partial(
        pl.kernel,
        out_type=jax.ShapeDtypeStruct((TOTAL, D), jnp.float32),
        mesh=mesh,
        scratch_types=[
            pltpu.VMEM((CHUNK,), jnp.int32),
            pltpu.VMEM((2, 32, D), jnp.float32),
            pltpu.VMEM((16,), jnp.int32),
            pltpu.VMEM((16,), jnp.int32),
            pltpu.VMEM((16, D), jnp.float32),
            pltpu.SemaphoreType.DMA((2,)),
            pltpu.SemaphoreType.DMA((2,)),
            pltpu.SemaphoreType.DMA,
        ],
    )(functools.partial(_sc_compact, TOTAL=TOTAL, D=D, CHUNK=CHUNK,
                        STRIDE=STRIDE, HALF0=HALF0, NSUB=NSUB, NPAD=NPAD))

    out = sck(x.reshape(B * T, D), src_all.reshape(TOTAL),
              means.reshape(B * NW, D), mdst.reshape(2 * NPAD),
              msrc.reshape(2 * NPAD))
    return out.reshape(B, OL, D)


# speedup vs baseline: 12.0466x; 1.0302x over previous
import functools
import math

import jax
import jax.numpy as jnp
from jax import lax
from jax.experimental import pallas as pl
from jax.experimental.pallas import tpu as pltpu
from jax.experimental.pallas import tpu_sc as plsc

_RATIO = 0.9
_W = 12

_NC = 2
_NS = 16


def _stats_kernel(x_hbm, maxstd_ref, means_ref, xbuf, stdacc, sem_in,
                  *, B, T, D, rem, W, nw_t, tiles, rows_a):
    rows_t = nw_t * W
    nsteps = tiles * B

    def tile_start(i):
        return min(((rem + i * rows_t) // 8) * 8, T - rows_a)

    def in_copy(s, slot):
        i, b = divmod(s, B)
        return pltpu.make_async_copy(
            x_hbm.at[b, pl.ds(tile_start(i), rows_a), :],
            xbuf.at[slot],
            sem_in.at[slot],
        )

    in_copy(0, 0).start()
    maxvals = []
    for s in range(nsteps):
        slot = s % 2
        i, b = divmod(s, B)
        if s + 1 < nsteps:
            in_copy(s + 1, (s + 1) % 2).start()
        in_copy(s, slot).wait()
        xb = xbuf[slot]
        mu = jnp.mean(xb, axis=1, keepdims=True)
        dd = xb - mu
        var = jnp.sum(dd * dd, axis=1, keepdims=True) * (1.0 / (D - 1))
        std = jnp.sqrt(var)
        if b == 0:
            stdacc[...] = std
        else:
            stdacc[...] = stdacc[...] + std

        a0 = tile_start(i)
        tok_r = lax.broadcasted_iota(jnp.int32, (nw_t, rows_a), 1) + a0
        w_id = lax.broadcasted_iota(jnp.int32, (nw_t, rows_a), 0) + i * nw_t
        valid = (tok_r >= rem) & ((tok_r - rem) // W == w_id)
        onehot = valid.astype(jnp.float32)
        wm = jnp.dot(onehot, xb, preferred_element_type=jnp.float32) * (1.0 / W)
        means_ref[b, i] = wm

        if b == B - 1:
            acc = stdacc[...] * (1.0 / B)
            tok_c = lax.broadcasted_iota(jnp.int32, (rows_a, nw_t), 0) + a0
            w_id2 = lax.broadcasted_iota(jnp.int32, (rows_a, nw_t), 1) + i * nw_t
            mask = (tok_c >= rem) & ((tok_c - rem) // W == w_id2)
            big = jnp.where(mask, jnp.broadcast_to(acc, (rows_a, nw_t)),
                            -jnp.inf)
            maxvals.append(jnp.max(big, axis=0))

    maxstd_ref[...] = jnp.stack(maxvals, axis=0)


def _select_kernel(mcol_ref, mrow_ref, src_ref, dst_ref, mdst_ref, msrc_ref,
                   *, B, T, NW, k, W, rem, L, WPB, CHUNK, STRIDE, EPAD):
    vc = mcol_ref[...]
    vr = mrow_ref[...]
    ii = lax.broadcasted_iota(jnp.int32, (NW, NW), 0)
    jj = lax.broadcasted_iota(jnp.int32, (NW, NW), 1)
    beats = ((vr < vc) | ((vr == vc) & (jj < ii))).astype(jnp.float32)
    rank_c = jnp.sum(beats, axis=1, keepdims=True)
    sel_c = (rank_c < k).astype(jnp.float32)
    size_c = jnp.where(sel_c > 0.5, 1.0, float(W))
    rank_r = (NW - 1) - jnp.sum(beats, axis=0, keepdims=True)
    sel_r = (rank_r < k).astype(jnp.float32)
    size_r = jnp.where(sel_r > 0.5, 1.0, float(W))
    size_rb = jnp.broadcast_to(size_r, (NW, NW))
    size_cb0 = jnp.broadcast_to(size_c, (NW, NW))
    sel_cb0 = jnp.broadcast_to(sel_c, (NW, NW))
    off_c = jnp.sum(jnp.where(jj < ii, size_rb, 0.0), axis=1, keepdims=True)
    off_row = jnp.sum(jnp.where(ii < jj, size_cb0, 0.0), axis=0, keepdims=True)
    c_row = jnp.sum(jnp.where(ii < jj, sel_cb0, 0.0), axis=0, keepdims=True)

    OL = rem + L
    NCOL = WPB * CHUNK
    colp = lax.broadcasted_iota(jnp.int32, (1, NCOL), 1)
    prow1 = (jnp.minimum(STRIDE * (colp // CHUNK), OL - CHUNK)
             + (colp % CHUNK))
    dst_ref[...] = prow1
    pp = jnp.broadcast_to(prow1.astype(jnp.float32), (NW, NCOL))
    ww = lax.broadcasted_iota(jnp.int32, (NW, NCOL), 0).astype(jnp.float32)
    ll = pp - float(rem)
    off_cb = jnp.broadcast_to(off_c, (NW, NCOL))
    size_cb = jnp.broadcast_to(size_c, (NW, NCOL))
    inwin = ((ll >= off_cb) & (ll < off_cb + size_cb)).astype(jnp.float32)
    contrib = inwin * (float(rem) + float(W) * ww + (ll - off_cb))
    f_row = jnp.sum(contrib, axis=0, keepdims=True)
    f_row = jnp.where(pp[0:1] < rem, pp[0:1], f_row)
    bb = lax.broadcasted_iota(jnp.int32, (B, NCOL), 0).astype(jnp.float32)
    src = f_row + bb * float(T)
    src_ref[...] = src.astype(jnp.int32)

    c_col = jnp.sum(jnp.where(jj < ii, jnp.broadcast_to(sel_r, (NW, NW)), 0.0),
                    axis=1, keepdims=True)
    ee = lax.broadcasted_iota(jnp.int32, (1, EPAD), 1).astype(jnp.float32)
    t_e = jnp.minimum(ee, float(k - 1))
    hit = (jnp.broadcast_to(sel_c, (NW, EPAD)) *
           (jnp.broadcast_to(c_col, (NW, EPAD)) ==
            jnp.broadcast_to(t_e, (NW, EPAD))).astype(jnp.float32))
    wsel = jnp.sum(hit * lax.broadcasted_iota(jnp.int32, (NW, EPAD), 0)
                   .astype(jnp.float32), axis=0, keepdims=True)
    dstl = float(rem) + jnp.sum(hit * jnp.broadcast_to(off_c, (NW, EPAD)),
                                axis=0, keepdims=True)
    mdst_ref[...] = dstl.astype(jnp.int32)
    bb2 = lax.broadcasted_iota(jnp.int32, (B, EPAD), 0).astype(jnp.float32)
    msrc_ref[...] = (jnp.broadcast_to(wsel, (B, EPAD)) +
                     bb2 * float(NW)).astype(jnp.int32)


def _sc_compact(x_hbm, src_hbm, dst_hbm, means_hbm, mdst_hbm, msrc_hbm,
                out_hbm, idx2, dst2, gbuf, mdstv, msrcv, mbuf,
                gsem, wsem, msem,
                *, OL, D, CHUNK, WPB, NSUB, EPAD):
    c = lax.axis_index("c")
    s = lax.axis_index("s")
    bq = c * (_NS // WPB) + s // WPB
    sq = s % WPB
    SUB = CHUNK // NSUB

    wslab = (bq * WPB + sq) * NSUB
    pltpu.sync_copy(src_hbm.at[pl.ds(wslab, NSUB), :], idx2)
    pltpu.sync_copy(dst_hbm.at[pl.ds(sq * NSUB, NSUB), :], dst2)

    def gather(it, slot):
        return pltpu.make_async_copy(
            x_hbm.at[idx2.at[it]], gbuf.at[slot], gsem.at[slot])

    def write(it, slot):
        return pltpu.make_async_copy(
            gbuf.at[slot], out_hbm.at[bq].at[dst2.at[it]], wsem.at[slot])

    gather(0, 0).start()
    for it in range(NSUB):
        slot = it % 2
        if it >= 1:
            write(it - 1, (it - 1) % 2).wait()
        if it + 1 < NSUB:
            gather(it + 1, (it + 1) % 2).start()
        gather(it, slot).wait()
        write(it, slot).start()
    write(NSUB - 1, (NSUB - 1) % 2).wait()

    plsc.subcore_barrier()

    pltpu.sync_copy(mdst_hbm.at[pl.ds(sq, 1), :], mdstv)
    pltpu.sync_copy(msrc_hbm.at[pl.ds(bq * WPB + sq, 1), :], msrcv)
    gm = pltpu.make_async_copy(means_hbm.at[msrcv.at[0]], mbuf, msem)
    gm.start()
    gm.wait()
    sm = pltpu.make_async_copy(mbuf, out_hbm.at[bq].at[mdstv.at[0]], msem)
    sm.start()
    sm.wait()


def kernel(x):
    B, T, D = x.shape
    W = _W
    k = math.floor((T - T * _RATIO) / W)
    rem = T % W
    NW = (T - rem) // W
    L = NW * W - k * (W - 1)
    OL = rem + L
    nw_t = max(d for d in range(1, min(NW, 32) + 1) if NW % d == 0)
    tiles = NW // nw_t
    rows_a = ((nw_t * W + rem) + 7) // 8 * 8

    maxstd, means = pl.pallas_call(
        functools.partial(_stats_kernel, B=B, T=T, D=D, rem=rem, W=W,
                          nw_t=nw_t, tiles=tiles, rows_a=rows_a),
        in_specs=[pl.BlockSpec(memory_space=pl.ANY)],
        out_specs=[pl.BlockSpec(memory_space=pltpu.VMEM),
                   pl.BlockSpec(memory_space=pltpu.VMEM)],
        out_shape=[jax.ShapeDtypeStruct((tiles, nw_t), jnp.float32),
                   jax.ShapeDtypeStruct((B, tiles, nw_t, D), jnp.float32)],
        scratch_shapes=[
            pltpu.VMEM((2, rows_a, D), jnp.float32),
            pltpu.VMEM((rows_a, 1), jnp.float32),
            pltpu.SemaphoreType.DMA((2,)),
        ],
    )(x)

    WPB = _NC * _NS // B
    EPAD = WPB * 16
    CHUNK = 512
    STRIDE = ((OL - CHUNK) + (WPB - 1) - 1) // (WPB - 1)
    NSUB = CHUNK // 32
    NCOL = WPB * CHUNK
    m = maxstd.reshape(NW)
    src_all, dstmap, mdst, msrc = pl.pallas_call(
        functools.partial(_select_kernel, B=B, T=T, NW=NW, k=k, W=W,
                          rem=rem, L=L, WPB=WPB, CHUNK=CHUNK, STRIDE=STRIDE,
                          EPAD=EPAD),
        out_shape=[jax.ShapeDtypeStruct((B, NCOL), jnp.int32),
                   jax.ShapeDtypeStruct((1, NCOL), jnp.int32),
                   jax.ShapeDtypeStruct((1, EPAD), jnp.int32),
                   jax.ShapeDtypeStruct((B, EPAD), jnp.int32)],
    )(m[:, None], m[None, :])

    mesh = plsc.VectorSubcoreMesh(core_axis_name="c", subcore_axis_name="s")
    sck = functools.partial(
        pl.kernel,
        out_type=jax.ShapeDtypeStruct((B, OL, D), jnp.float32),
        mesh=mesh,
        scratch_types=[
            pltpu.VMEM((NSUB, 32), jnp.int32),
            pltpu.VMEM((NSUB, 32), jnp.int32),
            pltpu.VMEM((2, 32, D), jnp.float32),
            pltpu.VMEM((1, 16), jnp.int32),
            pltpu.VMEM((1, 16), jnp.int32),
            pltpu.VMEM((16, D), jnp.float32),
            pltpu.SemaphoreType.DMA((2,)),
            pltpu.SemaphoreType.DMA((2,)),
            pltpu.SemaphoreType.DMA,
        ],
    )(functools.partial(_sc_compact, OL=OL, D=D, CHUNK=CHUNK,
                        WPB=WPB, NSUB=NSUB, EPAD=EPAD))

    return sck(x.reshape(B * T, D), src_all.reshape(B * WPB * NSUB, 32),
               dstmap.reshape(WPB * NSUB, 32),
               means.reshape(B * NW, D), mdst.reshape(WPB, 16),
               msrc.reshape(B * WPB, 16))
